# baseline (speedup 1.0000x reference)
import functools

import jax
import jax.numpy as jnp
from jax import lax
from jax.experimental import pallas as pl
from jax.experimental.pallas import tpu as pltpu
from jax.experimental.pallas import tpu_sc as plsc

VOCAB = 1000000
SEQ_LEN = 200
EMBED_DIM = 32
BATCH = 4096

NC = 2
NS = 16
NW = NC * NS

ST = SEQ_LEN // 8
BT = BATCH // 128
NCHUNKS = ST
ROWS = 8 * 128


def _body(idx_hbm, tok_hbm, pos_hbm, out_hbm,
          idx0, idx1, gbuf0, gbuf1, tbuf, pos_v, si0, si1, sg0, sg1, ss):
    w = lax.axis_index("s") * NC + lax.axis_index("c")
    pltpu.sync_copy(pos_hbm, pos_v)

    idxs, gbufs = (idx0, idx1), (gbuf0, gbuf1)
    sis, sgs = (si0, si1), (sg0, sg1)

    lane = lax.iota(jnp.int32, 16)
    dhi0 = lane // 8
    dhi1 = dhi0 + 2
    dlin = (lane % 8) * 128

    def start_idx(g, b):
        ts = jnp.minimum(g, ST - 1)
        pltpu.async_copy(idx_hbm.at[ts, w], idxs[b], sis[b])

    def wait_idx(b):
        pltpu.make_async_copy(idx_hbm.at[0, 0], idxs[b], sis[b]).wait()

    def fire_gather(b):
        for sl in range(8):
            pltpu.async_copy(
                tok_hbm.at[idxs[b].at[sl]],
                gbufs[b].at[pl.ds(sl * 128, 128)],
                sgs[b],
            )

    def wait_gather(b):
        pltpu.make_async_copy(tok_hbm.at[pl.ds(0, ROWS)], gbufs[b], sgs[b]).wait()

    def start_store(g):
        pltpu.async_copy(tbuf, out_hbm.at[g, :, w, :], ss)

    def wait_store():
        pltpu.make_async_copy(tbuf, out_hbm.at[0, :, 0, :], ss).wait()

    def work(g, b):
        gv = gbufs[b]

        def persl(sl, c):
            s = g * 8 + sl
            pos0 = pos_v[s, pl.ds(0, 16)]
            pos1 = pos_v[s, pl.ds(16, 16)]
            row0 = dhi0 + sl * 4
            row1 = dhi1 + sl * 4
            t0 = sl * 128

            def token(bl, c2):
                t = t0 + bl
                col = dlin + bl
                plsc.store_scatter(tbuf, [row0, col], gv[t, pl.ds(0, 16)] + pos0)
                plsc.store_scatter(tbuf, [row1, col], gv[t, pl.ds(16, 16)] + pos1)
                return c2

            lax.fori_loop(0, 128, token, 0, unroll=16)
            return c

        lax.fori_loop(0, 8, persl, 0)

    start_idx(0, 0)
    wait_idx(0)
    fire_gather(0)
    start_idx(1, 1)
    wait_idx(1)
    fire_gather(1)
    wait_gather(0)
    start_idx(2, 0)
    work(0, 0)
    start_store(0)

    @pl.loop(1, NCHUNKS - 2, step=2)
    def _(g0):
        for b, g_off in ((1, 0), (0, 1)):
            g = g0 + g_off
            ob = 1 - b
            wait_idx(ob)
            fire_gather(ob)
            wait_gather(b)
            start_idx(g + 2, b)
            wait_store()
            work(g, b)
            start_store(g)

    wait_idx(0)
    fire_gather(0)
    wait_gather(1)
    wait_store()
    work(NCHUNKS - 2, 1)
    start_store(NCHUNKS - 2)

    wait_gather(0)
    wait_store()
    work(NCHUNKS - 1, 0)
    start_store(NCHUNKS - 1)
    wait_store()


@jax.jit
def _run(idx4, token_table, pos_table):
    mesh = plsc.VectorSubcoreMesh(
        core_axis_name="c", subcore_axis_name="s", num_cores=NC, num_subcores=NS
    )
    return pl.kernel(
        _body,
        out_type=jax.ShapeDtypeStruct((ST, 32, BT, 1024), jnp.float32),
        mesh=mesh,
        scratch_types=[
            pltpu.VMEM((8, 128), jnp.int32),
            pltpu.VMEM((8, 128), jnp.int32),
            pltpu.VMEM((ROWS, EMBED_DIM), jnp.float32),
            pltpu.VMEM((ROWS, EMBED_DIM), jnp.float32),
            pltpu.VMEM((32, 1024), jnp.float32),
            pltpu.VMEM((SEQ_LEN, EMBED_DIM), jnp.float32),
            pltpu.SemaphoreType.DMA,
            pltpu.SemaphoreType.DMA,
            pltpu.SemaphoreType.DMA,
            pltpu.SemaphoreType.DMA,
            pltpu.SemaphoreType.DMA,
        ],
        compiler_params=pltpu.CompilerParams(use_tc_tiling_on_sc=False, needs_layout_passes=False),
    )(idx4, token_table, pos_table)


def kernel(inputs, token_table, pos_table):
    idx4 = (inputs.T.reshape(ST, 8, BT, 128).transpose(0, 2, 1, 3)
            .astype(jnp.int32))
    out5 = _run(idx4, token_table, pos_table)
    out = (out5.reshape(ST, 8, 4, BT, 8, 128)
           .transpose(3, 5, 0, 1, 2, 4)
           .reshape(BATCH, SEQ_LEN, EMBED_DIM))
    return out


# speedup vs baseline: 1.2326x; 1.2326x over previous
import functools

import jax
import jax.numpy as jnp
from jax import lax
from jax.experimental import pallas as pl
from jax.experimental.pallas import tpu as pltpu
from jax.experimental.pallas import tpu_sc as plsc

VOCAB = 1000000
SEQ_LEN = 200
EMBED_DIM = 32
BATCH = 4096

NC = 2
NS = 16
NW = NC * NS

ST = SEQ_LEN // 8
BT = BATCH // 128
NCHUNKS = ST
ROWS = 8 * 128


def _body(idx_hbm, tok_hbm, pos_hbm, out_hbm,
          idx0, idx1, gbuf0, gbuf1, tbuf, pos_v, si0, si1, sg0, sg1, ss):
    w = lax.axis_index("s") * NC + lax.axis_index("c")
    pltpu.sync_copy(pos_hbm, pos_v)

    idxs, gbufs = (idx0, idx1), (gbuf0, gbuf1)
    sis, sgs = (si0, si1), (sg0, sg1)

    lane = lax.iota(jnp.int32, 16)
    dhi0 = lane // 8
    dhi1 = dhi0 + 2
    dlin = (lane % 8) * 128

    def start_idx(g, b):
        ts = jnp.minimum(g, ST - 1)
        pltpu.async_copy(idx_hbm.at[ts, w], idxs[b], sis[b])

    def wait_idx(b):
        pltpu.make_async_copy(idx_hbm.at[0, 0], idxs[b], sis[b]).wait()

    def fire_gather(b):
        for sl in range(8):
            pltpu.async_copy(
                tok_hbm.at[idxs[b].at[sl]],
                gbufs[b].at[pl.ds(sl * 128, 128)],
                sgs[b],
            )

    def wait_gather(b):
        pltpu.make_async_copy(tok_hbm.at[pl.ds(0, ROWS)], gbufs[b], sgs[b]).wait()

    def start_store(g):
        pltpu.async_copy(tbuf, out_hbm.at[g, :, w, :], ss)

    def wait_store():
        pltpu.make_async_copy(tbuf, out_hbm.at[0, :, 0, :], ss).wait()

    def work(g, b):
        gv = gbufs[b]

        def persl(sl, c):
            s = g * 8 + sl
            pos0 = pos_v[s, pl.ds(0, 16)]
            pos1 = pos_v[s, pl.ds(16, 16)]
            row0 = dhi0 + sl * 4
            row1 = dhi1 + sl * 4
            t0 = sl * 128

            @plsc.parallel_loop(0, 128, unroll=8)
            def token(bl):
                t = t0 + bl
                col = dlin + bl
                plsc.store_scatter(tbuf, [row0, col], gv[t, pl.ds(0, 16)] + pos0)
                plsc.store_scatter(tbuf, [row1, col], gv[t, pl.ds(16, 16)] + pos1)

            return c

        lax.fori_loop(0, 8, persl, 0)

    start_idx(0, 0)
    wait_idx(0)
    fire_gather(0)
    start_idx(1, 1)
    wait_idx(1)
    fire_gather(1)
    wait_gather(0)
    start_idx(2, 0)
    work(0, 0)
    start_store(0)

    @pl.loop(1, NCHUNKS - 2, step=2)
    def _(g0):
        for b, g_off in ((1, 0), (0, 1)):
            g = g0 + g_off
            ob = 1 - b
            wait_idx(ob)
            fire_gather(ob)
            wait_gather(b)
            start_idx(g + 2, b)
            wait_store()
            work(g, b)
            start_store(g)

    wait_idx(0)
    fire_gather(0)
    wait_gather(1)
    wait_store()
    work(NCHUNKS - 2, 1)
    start_store(NCHUNKS - 2)

    wait_gather(0)
    wait_store()
    work(NCHUNKS - 1, 0)
    start_store(NCHUNKS - 1)
    wait_store()


@jax.jit
def _run(idx4, token_table, pos_table):
    mesh = plsc.VectorSubcoreMesh(
        core_axis_name="c", subcore_axis_name="s", num_cores=NC, num_subcores=NS
    )
    return pl.kernel(
        _body,
        out_type=jax.ShapeDtypeStruct((ST, 32, BT, 1024), jnp.float32),
        mesh=mesh,
        scratch_types=[
            pltpu.VMEM((8, 128), jnp.int32),
            pltpu.VMEM((8, 128), jnp.int32),
            pltpu.VMEM((ROWS, EMBED_DIM), jnp.float32),
            pltpu.VMEM((ROWS, EMBED_DIM), jnp.float32),
            pltpu.VMEM((32, 1024), jnp.float32),
            pltpu.VMEM((SEQ_LEN, EMBED_DIM), jnp.float32),
            pltpu.SemaphoreType.DMA,
            pltpu.SemaphoreType.DMA,
            pltpu.SemaphoreType.DMA,
            pltpu.SemaphoreType.DMA,
            pltpu.SemaphoreType.DMA,
        ],
        compiler_params=pltpu.CompilerParams(use_tc_tiling_on_sc=False, needs_layout_passes=False),
    )(idx4, token_table, pos_table)


def kernel(inputs, token_table, pos_table):
    idx4 = (inputs.T.reshape(ST, 8, BT, 128).transpose(0, 2, 1, 3)
            .astype(jnp.int32))
    out5 = _run(idx4, token_table, pos_table)
    out = (out5.reshape(ST, 8, 4, BT, 8, 128)
           .transpose(3, 5, 0, 1, 2, 4)
           .reshape(BATCH, SEQ_LEN, EMBED_DIM))
    return out
